# Initial kernel scaffold; baseline (speedup 1.0000x reference)
import functools

import jax
import jax.numpy as jnp
from jax import lax
from jax.experimental import pallas as pl
from jax.experimental.pallas import tpu as pltpu
from jax.experimental.pallas import tpu_sc as plsc

N = 10000
E = 160000
H = 256
OUT = 128
IB = 128
ROWS = E // IB
RPT = ROWS // 16
NPT = N // 16


def _mm3_body(relu_in, fh, xu_ref, xi_ref, wf_ref, wr_ref, wrb_ref,
              f_lo, f_hi, r_lo, r_hi, rb_lo, rb_hi):
    xu = xu_ref[...]
    xi = xi_ref[...]
    if relu_in:
        xu = jnp.maximum(xu, 0.0)
        xi = jnp.maximum(xi, 0.0)
    mf = jnp.dot(xu, wf_ref[...], preferred_element_type=jnp.float32)
    mr = jnp.dot(xu, wr_ref[...], preferred_element_type=jnp.float32)
    mrb = jnp.dot(xi, wrb_ref[...], preferred_element_type=jnp.float32)
    f_lo[...] = mf[:, :fh]
    f_hi[...] = mf[:, fh:]
    r_lo[...] = mr[:, :fh]
    r_hi[...] = mr[:, fh:]
    rb_lo[...] = mrb[:, :fh]
    rb_hi[...] = mrb[:, fh:]


def _make_mm3(fout, relu_in):
    fh = fout // 2
    bm = 1000
    grid = (N // bm,)
    half = jax.ShapeDtypeStruct((N, fh), jnp.float32)
    return pl.pallas_call(
        functools.partial(_mm3_body, relu_in, fh),
        grid=grid,
        in_specs=[
            pl.BlockSpec((bm, H), lambda i: (i, 0)),
            pl.BlockSpec((bm, H), lambda i: (i, 0)),
            pl.BlockSpec((H, fout), lambda i: (0, 0)),
            pl.BlockSpec((H, fout), lambda i: (0, 0)),
            pl.BlockSpec((H, fout), lambda i: (0, 0)),
        ],
        out_specs=[pl.BlockSpec((bm, fh), lambda i: (i, 0))] * 6,
        out_shape=[half] * 6,
        compiler_params=pltpu.CompilerParams(
            dimension_semantics=("parallel",)),
    )


def _relu2_body(a_ref, b_ref, oa_ref, ob_ref):
    oa_ref[...] = jnp.maximum(a_ref[...], 0.0)
    ob_ref[...] = jnp.maximum(b_ref[...], 0.0)


def _make_relu2():
    bm = 1000
    shp = jax.ShapeDtypeStruct((N, OUT), jnp.float32)
    return pl.pallas_call(
        _relu2_body,
        grid=(N // bm,),
        in_specs=[pl.BlockSpec((bm, OUT), lambda i: (i, 0))] * 2,
        out_specs=[pl.BlockSpec((bm, OUT), lambda i: (i, 0))] * 2,
        out_shape=[shp, shp],
        compiler_params=pltpu.CompilerParams(
            dimension_semantics=("parallel",)),
    )


def _make_spmm(n_rel, feat):
    fh = feat // 2
    mesh = plsc.VectorSubcoreMesh(core_axis_name="c", subcore_axis_name="s",
                                  num_cores=2)

    def body(*refs):
        h_refs = refs[:2 * n_rel]
        ei_refs = refs[2 * n_rel:3 * n_rel]
        zeros_ref = refs[3 * n_rel]
        out_ref = refs[3 * n_rel + 1]
        sidx, didx, rows, acc = refs[3 * n_rel + 2:]

        c = lax.axis_index("c")
        s = lax.axis_index("s")

        pltpu.sync_copy(zeros_ref.at[pl.ds(s * NPT, NPT), pl.ds(0, fh)],
                        acc.at[pl.ds(s * NPT, NPT), :])
        plsc.subcore_barrier()

        def process(h_ref, ei_ref):
            base = s * RPT
            pltpu.sync_copy(ei_ref.at[0, pl.ds(base, RPT), :],
                            sidx.at[pl.ds(0, RPT), :])
            pltpu.sync_copy(ei_ref.at[1, pl.ds(base, RPT), :],
                            didx.at[pl.ds(0, RPT), :])

            @pl.when(s == 15)
            def _():
                pltpu.sync_copy(ei_ref.at[0, pl.ds(16 * RPT, 2), :],
                                sidx.at[pl.ds(RPT, 2), :])
                pltpu.sync_copy(ei_ref.at[1, pl.ds(16 * RPT, 2), :],
                                didx.at[pl.ds(RPT, 2), :])

            nt = RPT + 2 * (s == 15)

            def step(t, carry):
                pltpu.sync_copy(h_ref.at[sidx.at[t]], rows)
                pltpu.sync_copy(rows, acc.at[didx.at[t]], add=True)
                return carry

            lax.fori_loop(0, nt, step, 0)

        @pl.when(c == 0)
        def _():
            for r in range(n_rel):
                process(h_refs[2 * r], ei_refs[r])

        @pl.when(c == 1)
        def _():
            for r in range(n_rel):
                process(h_refs[2 * r + 1], ei_refs[r])

        plsc.subcore_barrier()
        pltpu.sync_copy(acc.at[pl.ds(s * NPT, NPT), :],
                        out_ref.at[pl.ds(s * NPT, NPT), pl.ds(c * fh, fh)])

    return pl.kernel(
        body,
        out_type=jax.ShapeDtypeStruct((N, feat), jnp.float32),
        mesh=mesh,
        scratch_types=[
            pltpu.VMEM((RPT + 2, IB), jnp.int32),
            pltpu.VMEM((RPT + 2, IB), jnp.int32),
            pltpu.VMEM((IB, fh), jnp.float32),
            pltpu.VMEM_SHARED((N, fh), jnp.float32),
        ],
    )


def kernel(x_user, x_item, W1_follows, W1_rates, W1_ratedby,
           W2_follows, W2_rates, W2_ratedby,
           ei_follows, ei_rates, ei_ratedby):
    ei_f = ei_follows.reshape(2, ROWS, IB)
    ei_r = ei_rates.reshape(2, ROWS, IB)
    ei_rb = ei_ratedby.reshape(2, ROWS, IB)
    zeros = jnp.zeros((N, 128), jnp.float32)

    mm1 = _make_mm3(H, relu_in=False)
    mm2 = _make_mm3(OUT, relu_in=True)
    spmm2_h = _make_spmm(2, H)
    spmm1_h = _make_spmm(1, H)
    spmm2_o = _make_spmm(2, OUT)
    spmm1_o = _make_spmm(1, OUT)
    relu2 = _make_relu2()

    hf_lo, hf_hi, hr_lo, hr_hi, hrb_lo, hrb_hi = mm1(
        x_user, x_item, W1_follows, W1_rates, W1_ratedby)
    hu1 = spmm2_h(hf_lo, hf_hi, hrb_lo, hrb_hi, ei_f, ei_rb, zeros)
    hi1 = spmm1_h(hr_lo, hr_hi, ei_r, zeros)

    gf_lo, gf_hi, gr_lo, gr_hi, grb_lo, grb_hi = mm2(
        hu1, hi1, W2_follows, W2_rates, W2_ratedby)
    hu2 = spmm2_o(gf_lo, gf_hi, grb_lo, grb_hi, ei_f, ei_rb, zeros)
    hi2 = spmm1_o(gr_lo, gr_hi, ei_r, zeros)

    return relu2(hu2, hi2)


# speedup vs baseline: 1.8645x; 1.8645x over previous
import functools

import jax
import jax.numpy as jnp
from jax import lax
from jax.experimental import pallas as pl
from jax.experimental.pallas import tpu as pltpu
from jax.experimental.pallas import tpu_sc as plsc

N = 10000
E = 160000
H = 256
OUT = 128
IB = 128
ROWS = 1280
EPAD = ROWS * IB - E
NA = N + 8
NPT = 624


def _mm1_body(xu_ref, xi_ref, wf_ref, wr_ref, wrb_ref,
              f_lo, f_hi, r_lo, r_hi, rb_lo, rb_hi):
    xu = xu_ref[...]
    xi = xi_ref[...]
    fh = H // 2
    mf = jnp.dot(xu, wf_ref[...], preferred_element_type=jnp.float32)
    mr = jnp.dot(xu, wr_ref[...], preferred_element_type=jnp.float32)
    mrb = jnp.dot(xi, wrb_ref[...], preferred_element_type=jnp.float32)
    f_lo[...] = mf[:, :fh]
    f_hi[...] = mf[:, fh:]
    r_lo[...] = mr[:, :fh]
    r_hi[...] = mr[:, fh:]
    rb_lo[...] = mrb[:, :fh]
    rb_hi[...] = mrb[:, fh:]


def _make_mm1():
    bm = 1000
    half = jax.ShapeDtypeStruct((N, H // 2), jnp.float32)
    return pl.pallas_call(
        _mm1_body,
        grid=(N // bm,),
        in_specs=[
            pl.BlockSpec((bm, H), lambda i: (i, 0)),
            pl.BlockSpec((bm, H), lambda i: (i, 0)),
            pl.BlockSpec((H, H), lambda i: (0, 0)),
            pl.BlockSpec((H, H), lambda i: (0, 0)),
            pl.BlockSpec((H, H), lambda i: (0, 0)),
        ],
        out_specs=[pl.BlockSpec((bm, H // 2), lambda i: (i, 0))] * 6,
        out_shape=[half] * 6,
        compiler_params=pltpu.CompilerParams(
            dimension_semantics=("parallel",)),
    )


def _mm2_body(xu_ref, xi_ref, wf_ref, wr_ref, wrb_ref, f_o, r_o, rb_o):
    xu = jnp.maximum(xu_ref[...], 0.0)
    xi = jnp.maximum(xi_ref[...], 0.0)
    f_o[...] = jnp.dot(xu, wf_ref[...], preferred_element_type=jnp.float32)
    r_o[...] = jnp.dot(xu, wr_ref[...], preferred_element_type=jnp.float32)
    rb_o[...] = jnp.dot(xi, wrb_ref[...], preferred_element_type=jnp.float32)


def _make_mm2():
    bm = 1000
    full = jax.ShapeDtypeStruct((N, OUT), jnp.float32)
    return pl.pallas_call(
        _mm2_body,
        grid=(N // bm,),
        in_specs=[
            pl.BlockSpec((bm, H), lambda i: (i, 0)),
            pl.BlockSpec((bm, H), lambda i: (i, 0)),
            pl.BlockSpec((H, OUT), lambda i: (0, 0)),
            pl.BlockSpec((H, OUT), lambda i: (0, 0)),
            pl.BlockSpec((H, OUT), lambda i: (0, 0)),
        ],
        out_specs=[pl.BlockSpec((bm, OUT), lambda i: (i, 0))] * 3,
        out_shape=[full] * 3,
        compiler_params=pltpu.CompilerParams(
            dimension_semantics=("parallel",)),
    )


def _merge_relu_body(a0_ref, a1_ref, b0_ref, b1_ref, oa_ref, ob_ref):
    oa_ref[...] = jnp.maximum(a0_ref[...] + a1_ref[...], 0.0)
    ob_ref[...] = jnp.maximum(b0_ref[...] + b1_ref[...], 0.0)


def _make_merge_relu():
    bm = 1000
    shp = jax.ShapeDtypeStruct((N, OUT), jnp.float32)
    return pl.pallas_call(
        _merge_relu_body,
        grid=(N // bm,),
        in_specs=[pl.BlockSpec((bm, OUT), lambda i: (i, 0))] * 4,
        out_specs=[pl.BlockSpec((bm, OUT), lambda i: (i, 0))] * 2,
        out_shape=[shp, shp],
        compiler_params=pltpu.CompilerParams(
            dimension_semantics=("parallel",)),
    )


_MESH = plsc.VectorSubcoreMesh(core_axis_name="c", subcore_axis_name="s",
                               num_cores=2)


def _clear_acc(zeros_ref, acc, s):
    pltpu.sync_copy(zeros_ref.at[pl.ds(s * NPT, NPT), :],
                    acc.at[pl.ds(s * NPT, NPT), :])

    @pl.when(s == 15)
    def _():
        pltpu.sync_copy(zeros_ref.at[pl.ds(16 * NPT, 16), :],
                        acc.at[pl.ds(16 * NPT, 16), :])


def _accumulate(h_ref, ei_ref, sidx, didx, rows, acc, base, nrows):
    pltpu.sync_copy(ei_ref.at[0, pl.ds(base, nrows), :],
                    sidx.at[pl.ds(0, nrows), :])
    pltpu.sync_copy(ei_ref.at[1, pl.ds(base, nrows), :],
                    didx.at[pl.ds(0, nrows), :])

    def step(t, carry):
        pltpu.sync_copy(h_ref.at[sidx.at[t]], rows)
        pltpu.sync_copy(rows, acc.at[didx.at[t]], add=True)
        return carry

    lax.fori_loop(0, nrows, step, 0)


def _make_spmm_featsplit(n_rel):
    fh = H // 2
    rpt = ROWS // 16

    def body(*refs):
        h_refs = refs[:2 * n_rel]
        ei_refs = refs[2 * n_rel:3 * n_rel]
        zeros_ref = refs[3 * n_rel]
        out_ref = refs[3 * n_rel + 1]
        sidx, didx, rows, acc = refs[3 * n_rel + 2:]

        c = lax.axis_index("c")
        s = lax.axis_index("s")

        _clear_acc(zeros_ref, acc, s)
        plsc.subcore_barrier()

        @pl.when(c == 0)
        def _():
            for r in range(n_rel):
                _accumulate(h_refs[2 * r], ei_refs[r], sidx, didx, rows,
                            acc, s * rpt, rpt)

        @pl.when(c == 1)
        def _():
            for r in range(n_rel):
                _accumulate(h_refs[2 * r + 1], ei_refs[r], sidx, didx, rows,
                            acc, s * rpt, rpt)

        plsc.subcore_barrier()
        pltpu.sync_copy(acc.at[pl.ds(s * NPT, NPT), :],
                        out_ref.at[pl.ds(s * NPT, NPT), pl.ds(c * fh, fh)])

        @pl.when(s == 15)
        def _():
            pltpu.sync_copy(acc.at[pl.ds(16 * NPT, 16), :],
                            out_ref.at[pl.ds(16 * NPT, 16), pl.ds(c * fh, fh)])

    return pl.kernel(
        body,
        out_type=jax.ShapeDtypeStruct((N, H), jnp.float32),
        mesh=_MESH,
        scratch_types=[
            pltpu.VMEM((ROWS // 16, IB), jnp.int32),
            pltpu.VMEM((ROWS // 16, IB), jnp.int32),
            pltpu.VMEM((IB, fh), jnp.float32),
            pltpu.VMEM_SHARED((NA, fh), jnp.float32),
        ],
    )


def _make_spmm_edgesplit(n_rel):
    rpw = ROWS // 32

    def body(*refs):
        h_refs = refs[:n_rel]
        ei_refs = refs[n_rel:2 * n_rel]
        zeros_ref = refs[2 * n_rel]
        out_refs = refs[2 * n_rel + 1:2 * n_rel + 3]
        sidx, didx, rows, acc = refs[2 * n_rel + 3:]

        c = lax.axis_index("c")
        s = lax.axis_index("s")
        w = c * 16 + s

        _clear_acc(zeros_ref, acc, s)
        plsc.subcore_barrier()

        for r in range(n_rel):
            _accumulate(h_refs[r], ei_refs[r], sidx, didx, rows,
                        acc, w * rpw, rpw)

        plsc.subcore_barrier()

        @pl.when(c == 0)
        def _():
            pltpu.sync_copy(acc.at[pl.ds(s * NPT, NPT), :],
                            out_refs[0].at[pl.ds(s * NPT, NPT), :])

            @pl.when(s == 15)
            def _():
                pltpu.sync_copy(acc.at[pl.ds(16 * NPT, 16), :],
                                out_refs[0].at[pl.ds(16 * NPT, 16), :])

        @pl.when(c == 1)
        def _():
            pltpu.sync_copy(acc.at[pl.ds(s * NPT, NPT), :],
                            out_refs[1].at[pl.ds(s * NPT, NPT), :])

            @pl.when(s == 15)
            def _():
                pltpu.sync_copy(acc.at[pl.ds(16 * NPT, 16), :],
                                out_refs[1].at[pl.ds(16 * NPT, 16), :])

    part = jax.ShapeDtypeStruct((N, OUT), jnp.float32)
    return pl.kernel(
        body,
        out_type=[part, part],
        mesh=_MESH,
        scratch_types=[
            pltpu.VMEM((ROWS // 32, IB), jnp.int32),
            pltpu.VMEM((ROWS // 32, IB), jnp.int32),
            pltpu.VMEM((IB, OUT), jnp.float32),
            pltpu.VMEM_SHARED((NA, OUT), jnp.float32),
        ],
    )


def _pad_edges(ei):
    pad = jnp.stack([jnp.zeros((EPAD,), jnp.int32),
                     jnp.full((EPAD,), N, jnp.int32)])
    return jnp.concatenate([ei.astype(jnp.int32), pad], axis=1).reshape(
        2, ROWS, IB)


def kernel(x_user, x_item, W1_follows, W1_rates, W1_ratedby,
           W2_follows, W2_rates, W2_ratedby,
           ei_follows, ei_rates, ei_ratedby):
    ei_f = _pad_edges(ei_follows)
    ei_r = _pad_edges(ei_rates)
    ei_rb = _pad_edges(ei_ratedby)
    zeros = jnp.zeros((N, 128), jnp.float32)

    mm1 = _make_mm1()
    mm2 = _make_mm2()
    spmm2_h = _make_spmm_featsplit(2)
    spmm1_h = _make_spmm_featsplit(1)
    spmm2_o = _make_spmm_edgesplit(2)
    spmm1_o = _make_spmm_edgesplit(1)
    merge_relu = _make_merge_relu()

    hf_lo, hf_hi, hr_lo, hr_hi, hrb_lo, hrb_hi = mm1(
        x_user, x_item, W1_follows, W1_rates, W1_ratedby)
    hu1 = spmm2_h(hf_lo, hf_hi, hrb_lo, hrb_hi, ei_f, ei_rb, zeros)
    hi1 = spmm1_h(hr_lo, hr_hi, ei_r, zeros)

    gf, gr, grb = mm2(hu1, hi1, W2_follows, W2_rates, W2_ratedby)
    hu2_p0, hu2_p1 = spmm2_o(gf, grb, ei_f, ei_rb, zeros)
    hi2_p0, hi2_p1 = spmm1_o(gr, ei_r, zeros)

    return merge_relu(hu2_p0, hu2_p1, hi2_p0, hi2_p1)


# speedup vs baseline: 4.8433x; 2.5977x over previous
import functools

import jax
import jax.numpy as jnp
from jax import lax
from jax.experimental import pallas as pl
from jax.experimental.pallas import tpu as pltpu
from jax.experimental.pallas import tpu_sc as plsc

N = 10000
E = 160000
H = 256
OUT = 128
IB = 128
ROWS = 1280
EPAD = ROWS * IB - E
NA = N + 16
NPT = 624
NBUF = 2
CH = 40


def _mm1_body(xu_ref, xi_ref, wf_ref, wr_ref, wrb_ref,
              f_lo, f_hi, r_lo, r_hi, rb_lo, rb_hi):
    xu = xu_ref[...]
    xi = xi_ref[...]
    fh = H // 2
    mf = jnp.dot(xu, wf_ref[...], preferred_element_type=jnp.float32)
    mr = jnp.dot(xu, wr_ref[...], preferred_element_type=jnp.float32)
    mrb = jnp.dot(xi, wrb_ref[...], preferred_element_type=jnp.float32)
    f_lo[...] = mf[:, :fh]
    f_hi[...] = mf[:, fh:]
    r_lo[...] = mr[:, :fh]
    r_hi[...] = mr[:, fh:]
    rb_lo[...] = mrb[:, :fh]
    rb_hi[...] = mrb[:, fh:]


def _make_mm1():
    bm = 1000
    half = jax.ShapeDtypeStruct((N, H // 2), jnp.float32)
    return pl.pallas_call(
        _mm1_body,
        grid=(N // bm,),
        in_specs=[
            pl.BlockSpec((bm, H), lambda i: (i, 0)),
            pl.BlockSpec((bm, H), lambda i: (i, 0)),
            pl.BlockSpec((H, H), lambda i: (0, 0)),
            pl.BlockSpec((H, H), lambda i: (0, 0)),
            pl.BlockSpec((H, H), lambda i: (0, 0)),
        ],
        out_specs=[pl.BlockSpec((bm, H // 2), lambda i: (i, 0))] * 6,
        out_shape=[half] * 6,
        compiler_params=pltpu.CompilerParams(
            dimension_semantics=("parallel",)),
    )


def _mm2_body(xu_ref, xi_ref, wf_ref, wr_ref, wrb_ref, f_o, r_o, rb_o):
    xu = jnp.maximum(xu_ref[...], 0.0)
    xi = jnp.maximum(xi_ref[...], 0.0)
    f_o[...] = jnp.dot(xu, wf_ref[...], preferred_element_type=jnp.float32)
    r_o[...] = jnp.dot(xu, wr_ref[...], preferred_element_type=jnp.float32)
    rb_o[...] = jnp.dot(xi, wrb_ref[...], preferred_element_type=jnp.float32)


def _make_mm2():
    bm = 1000
    full = jax.ShapeDtypeStruct((N, OUT), jnp.float32)
    return pl.pallas_call(
        _mm2_body,
        grid=(N // bm,),
        in_specs=[
            pl.BlockSpec((bm, H), lambda i: (i, 0)),
            pl.BlockSpec((bm, H), lambda i: (i, 0)),
            pl.BlockSpec((H, OUT), lambda i: (0, 0)),
            pl.BlockSpec((H, OUT), lambda i: (0, 0)),
            pl.BlockSpec((H, OUT), lambda i: (0, 0)),
        ],
        out_specs=[pl.BlockSpec((bm, OUT), lambda i: (i, 0))] * 3,
        out_shape=[full] * 3,
        compiler_params=pltpu.CompilerParams(
            dimension_semantics=("parallel",)),
    )


def _merge_relu_body(a0_ref, a1_ref, b0_ref, b1_ref, oa_ref, ob_ref):
    oa_ref[...] = jnp.maximum(a0_ref[...] + a1_ref[...], 0.0)
    ob_ref[...] = jnp.maximum(b0_ref[...] + b1_ref[...], 0.0)


def _make_merge_relu():
    bm = 1000
    shp = jax.ShapeDtypeStruct((N, OUT), jnp.float32)
    return pl.pallas_call(
        _merge_relu_body,
        grid=(N // bm,),
        in_specs=[pl.BlockSpec((bm, OUT), lambda i: (i, 0))] * 4,
        out_specs=[pl.BlockSpec((bm, OUT), lambda i: (i, 0))] * 2,
        out_shape=[shp, shp],
        compiler_params=pltpu.CompilerParams(
            dimension_semantics=("parallel",)),
    )


_MESH = plsc.VectorSubcoreMesh(core_axis_name="c", subcore_axis_name="s",
                               num_cores=2)


def _clear_acc(zeros_ref, acc, s):
    pltpu.sync_copy(zeros_ref.at[pl.ds(s * NPT, NPT), :],
                    acc.at[pl.ds(s * NPT, NPT), :])

    @pl.when(s == 15)
    def _():
        pltpu.sync_copy(zeros_ref.at[pl.ds(16 * NPT, 16), :],
                        acc.at[pl.ds(16 * NPT, 16), :])


def _accumulate(h_ref, ei_ref, sidx, didx, rows, gsems, ssems, acc,
                base, nrows):
    ngrp = CH // NBUF

    def chunk(ci, carry):
        cbase = base + ci * CH
        pltpu.sync_copy(ei_ref.at[0, pl.ds(cbase, CH), :], sidx)
        pltpu.sync_copy(ei_ref.at[1, pl.ds(cbase, CH), :], didx)

        for b in range(NBUF):
            pltpu.async_copy(h_ref.at[sidx.at[b]], rows[b], gsems[b])

        def group(g, c2):
            t0 = g * NBUF
            for b in range(NBUF):
                pltpu.make_async_copy(h_ref.at[sidx.at[t0 + b]], rows[b],
                                      gsems[b]).wait()
                pltpu.async_copy(rows[b], acc.at[didx.at[t0 + b]], ssems[b],
                                 add=True)
            for b in range(NBUF):
                pltpu.make_async_copy(rows[b], acc.at[didx.at[t0 + b]],
                                      ssems[b]).wait()

                @pl.when(g + 1 < ngrp)
                def _():
                    pltpu.async_copy(h_ref.at[sidx.at[t0 + NBUF + b]],
                                     rows[b], gsems[b])
            return c2

        lax.fori_loop(0, ngrp, group, 0)
        return carry

    lax.fori_loop(0, nrows // CH, chunk, 0)


def _make_spmm_featsplit(n_rel):
    fh = H // 2
    rpt = ROWS // 16

    def body(*refs):
        h_refs = refs[:2 * n_rel]
        ei_refs = refs[2 * n_rel:3 * n_rel]
        zeros_ref = refs[3 * n_rel]
        out_ref = refs[3 * n_rel + 1]
        sidx = refs[3 * n_rel + 2]
        didx = refs[3 * n_rel + 3]
        rows = refs[3 * n_rel + 4:3 * n_rel + 4 + NBUF]
        acc = refs[3 * n_rel + 4 + NBUF]
        gsems = refs[3 * n_rel + 5 + NBUF:3 * n_rel + 5 + 2 * NBUF]
        ssems = refs[3 * n_rel + 5 + 2 * NBUF:]

        c = lax.axis_index("c")
        s = lax.axis_index("s")

        _clear_acc(zeros_ref, acc, s)
        plsc.subcore_barrier()

        @pl.when(c == 0)
        def _():
            for r in range(n_rel):
                _accumulate(h_refs[2 * r], ei_refs[r], sidx, didx, rows,
                            gsems, ssems, acc, s * rpt, rpt)

        @pl.when(c == 1)
        def _():
            for r in range(n_rel):
                _accumulate(h_refs[2 * r + 1], ei_refs[r], sidx, didx, rows,
                            gsems, ssems, acc, s * rpt, rpt)

        plsc.subcore_barrier()
        pltpu.sync_copy(acc.at[pl.ds(s * NPT, NPT), :],
                        out_ref.at[pl.ds(s * NPT, NPT), pl.ds(c * fh, fh)])

        @pl.when(s == 15)
        def _():
            pltpu.sync_copy(acc.at[pl.ds(16 * NPT, 16), :],
                            out_ref.at[pl.ds(16 * NPT, 16), pl.ds(c * fh, fh)])

    return pl.kernel(
        body,
        out_type=jax.ShapeDtypeStruct((N, H), jnp.float32),
        mesh=_MESH,
        scratch_types=(
            [pltpu.VMEM((CH, IB), jnp.int32)] * 2 +
            [pltpu.VMEM((IB, fh), jnp.float32)] * NBUF +
            [pltpu.VMEM_SHARED((NA, fh), jnp.float32)] +
            [pltpu.SemaphoreType.DMA] * (2 * NBUF)
        ),
    )


def _make_spmm_edgesplit(n_rel):
    rpw = ROWS // 32

    def body(*refs):
        h_refs = refs[:n_rel]
        ei_refs = refs[n_rel:2 * n_rel]
        zeros_ref = refs[2 * n_rel]
        out_refs = refs[2 * n_rel + 1:2 * n_rel + 3]
        sidx = refs[2 * n_rel + 3]
        didx = refs[2 * n_rel + 4]
        rows = refs[2 * n_rel + 5:2 * n_rel + 5 + NBUF]
        acc = refs[2 * n_rel + 5 + NBUF]
        gsems = refs[2 * n_rel + 6 + NBUF:2 * n_rel + 6 + 2 * NBUF]
        ssems = refs[2 * n_rel + 6 + 2 * NBUF:]

        c = lax.axis_index("c")
        s = lax.axis_index("s")
        w = c * 16 + s

        _clear_acc(zeros_ref, acc, s)
        plsc.subcore_barrier()

        for r in range(n_rel):
            _accumulate(h_refs[r], ei_refs[r], sidx, didx, rows,
                        gsems, ssems, acc, w * rpw, rpw)

        plsc.subcore_barrier()

        @pl.when(c == 0)
        def _():
            pltpu.sync_copy(acc.at[pl.ds(s * NPT, NPT), :],
                            out_refs[0].at[pl.ds(s * NPT, NPT), :])

            @pl.when(s == 15)
            def _():
                pltpu.sync_copy(acc.at[pl.ds(16 * NPT, 16), :],
                                out_refs[0].at[pl.ds(16 * NPT, 16), :])

        @pl.when(c == 1)
        def _():
            pltpu.sync_copy(acc.at[pl.ds(s * NPT, NPT), :],
                            out_refs[1].at[pl.ds(s * NPT, NPT), :])

            @pl.when(s == 15)
            def _():
                pltpu.sync_copy(acc.at[pl.ds(16 * NPT, 16), :],
                                out_refs[1].at[pl.ds(16 * NPT, 16), :])

    part = jax.ShapeDtypeStruct((N, OUT), jnp.float32)
    return pl.kernel(
        body,
        out_type=[part, part],
        mesh=_MESH,
        scratch_types=(
            [pltpu.VMEM((CH, IB), jnp.int32)] * 2 +
            [pltpu.VMEM((IB, OUT), jnp.float32)] * NBUF +
            [pltpu.VMEM_SHARED((NA, OUT), jnp.float32)] +
            [pltpu.SemaphoreType.DMA] * (2 * NBUF)
        ),
    )


def _pad_edges(ei):
    r = jnp.arange(EPAD, dtype=jnp.int32)
    pad = jnp.stack([r % N, N + (r % (NA - N))])
    return jnp.concatenate([ei.astype(jnp.int32), pad], axis=1).reshape(
        2, ROWS, IB)


def kernel(x_user, x_item, W1_follows, W1_rates, W1_ratedby,
           W2_follows, W2_rates, W2_ratedby,
           ei_follows, ei_rates, ei_ratedby):
    ei_f = _pad_edges(ei_follows)
    ei_r = _pad_edges(ei_rates)
    ei_rb = _pad_edges(ei_ratedby)
    zeros = jnp.zeros((N, 128), jnp.float32)

    mm1 = _make_mm1()
    mm2 = _make_mm2()
    spmm2_h = _make_spmm_featsplit(2)
    spmm1_h = _make_spmm_featsplit(1)
    spmm2_o = _make_spmm_edgesplit(2)
    spmm1_o = _make_spmm_edgesplit(1)
    merge_relu = _make_merge_relu()

    hf_lo, hf_hi, hr_lo, hr_hi, hrb_lo, hrb_hi = mm1(
        x_user, x_item, W1_follows, W1_rates, W1_ratedby)
    hu1 = spmm2_h(hf_lo, hf_hi, hrb_lo, hrb_hi, ei_f, ei_rb, zeros)
    dep1 = (hu1[0, 0] * 0.0).astype(jnp.int32)
    hi1 = spmm1_h(hr_lo, hr_hi, ei_r + dep1, zeros)

    gf, gr, grb = mm2(hu1, hi1, W2_follows, W2_rates, W2_ratedby)
    hu2_p0, hu2_p1 = spmm2_o(gf, grb, ei_f, ei_rb, zeros)
    dep2 = (hu2_p0[0, 0] * 0.0).astype(jnp.int32)
    hi2_p0, hi2_p1 = spmm1_o(gr, ei_r + dep2, zeros)

    return merge_relu(hu2_p0, hu2_p1, hi2_p0, hi2_p1)


# speedup vs baseline: 4.9836x; 1.0290x over previous
import jax
import jax.numpy as jnp
from jax import lax
from jax.experimental import pallas as pl
from jax.experimental.pallas import tpu as pltpu
from jax.experimental.pallas import tpu_sc as plsc

N = 10000
E = 160000
H = 256
OUT = 128
IB = 128
ROWS = 1280
EPAD = ROWS * IB - E
NA = N + 16
NPT = 624
NBUF = 2
CH = 40


def _mm1a_body(xu_ref, xi_ref, wf_ref, wrb_ref,
               f_lo, f_hi, rb_lo, rb_hi):
    fh = H // 2
    mf = jnp.dot(xu_ref[...], wf_ref[...], preferred_element_type=jnp.float32)
    mrb = jnp.dot(xi_ref[...], wrb_ref[...],
                  preferred_element_type=jnp.float32)
    f_lo[...] = mf[:, :fh]
    f_hi[...] = mf[:, fh:]
    rb_lo[...] = mrb[:, :fh]
    rb_hi[...] = mrb[:, fh:]


def _make_mm1a():
    bm = 1000
    half = jax.ShapeDtypeStruct((N, H // 2), jnp.float32)
    return pl.pallas_call(
        _mm1a_body,
        grid=(N // bm,),
        in_specs=[
            pl.BlockSpec((bm, H), lambda i: (i, 0)),
            pl.BlockSpec((bm, H), lambda i: (i, 0)),
            pl.BlockSpec((H, H), lambda i: (0, 0)),
            pl.BlockSpec((H, H), lambda i: (0, 0)),
        ],
        out_specs=[pl.BlockSpec((bm, H // 2), lambda i: (i, 0))] * 4,
        out_shape=[half] * 4,
        compiler_params=pltpu.CompilerParams(
            dimension_semantics=("parallel",)),
    )


def _mm1b_body(xu_ref, wr_ref, r_lo, r_hi):
    fh = H // 2
    mr = jnp.dot(xu_ref[...], wr_ref[...], preferred_element_type=jnp.float32)
    r_lo[...] = mr[:, :fh]
    r_hi[...] = mr[:, fh:]


def _make_mm1b():
    bm = 1000
    half = jax.ShapeDtypeStruct((N, H // 2), jnp.float32)
    return pl.pallas_call(
        _mm1b_body,
        grid=(N // bm,),
        in_specs=[
            pl.BlockSpec((bm, H), lambda i: (i, 0)),
            pl.BlockSpec((H, H), lambda i: (0, 0)),
        ],
        out_specs=[pl.BlockSpec((bm, H // 2), lambda i: (i, 0))] * 2,
        out_shape=[half] * 2,
        compiler_params=pltpu.CompilerParams(
            dimension_semantics=("parallel",)),
    )


def _mm2a_body(xu_ref, wf_ref, wr_ref, f_o, r_o):
    xu = jnp.maximum(xu_ref[...], 0.0)
    f_o[...] = jnp.dot(xu, wf_ref[...], preferred_element_type=jnp.float32)
    r_o[...] = jnp.dot(xu, wr_ref[...], preferred_element_type=jnp.float32)


def _make_mm2a():
    bm = 1000
    full = jax.ShapeDtypeStruct((N, OUT), jnp.float32)
    return pl.pallas_call(
        _mm2a_body,
        grid=(N // bm,),
        in_specs=[
            pl.BlockSpec((bm, H), lambda i: (i, 0)),
            pl.BlockSpec((H, OUT), lambda i: (0, 0)),
            pl.BlockSpec((H, OUT), lambda i: (0, 0)),
        ],
        out_specs=[pl.BlockSpec((bm, OUT), lambda i: (i, 0))] * 2,
        out_shape=[full] * 2,
        compiler_params=pltpu.CompilerParams(
            dimension_semantics=("parallel",)),
    )


def _mm2b_body(xi_ref, wrb_ref, rb_o):
    xi = jnp.maximum(xi_ref[...], 0.0)
    rb_o[...] = jnp.dot(xi, wrb_ref[...], preferred_element_type=jnp.float32)


def _make_mm2b():
    bm = 1000
    full = jax.ShapeDtypeStruct((N, OUT), jnp.float32)
    return pl.pallas_call(
        _mm2b_body,
        grid=(N // bm,),
        in_specs=[
            pl.BlockSpec((bm, H), lambda i: (i, 0)),
            pl.BlockSpec((H, OUT), lambda i: (0, 0)),
        ],
        out_specs=[pl.BlockSpec((bm, OUT), lambda i: (i, 0))],
        out_shape=[full],
        compiler_params=pltpu.CompilerParams(
            dimension_semantics=("parallel",)),
    )


def _merge_relu_body(a0_ref, a1_ref, b0_ref, b1_ref, oa_ref, ob_ref):
    oa_ref[...] = jnp.maximum(a0_ref[...] + a1_ref[...], 0.0)
    ob_ref[...] = jnp.maximum(b0_ref[...] + b1_ref[...], 0.0)


def _make_merge_relu():
    bm = 1000
    shp = jax.ShapeDtypeStruct((N, OUT), jnp.float32)
    return pl.pallas_call(
        _merge_relu_body,
        grid=(N // bm,),
        in_specs=[pl.BlockSpec((bm, OUT), lambda i: (i, 0))] * 4,
        out_specs=[pl.BlockSpec((bm, OUT), lambda i: (i, 0))] * 2,
        out_shape=[shp, shp],
        compiler_params=pltpu.CompilerParams(
            dimension_semantics=("parallel",)),
    )


_MESH = plsc.VectorSubcoreMesh(core_axis_name="c", subcore_axis_name="s",
                               num_cores=2)


def _clear_acc(zbuf, acc, s):
    zero = jnp.zeros((16,), jnp.float32)

    def zrow(r, carry):
        for j in range(8):
            zbuf[r, pl.ds(16 * j, 16)] = zero
        return carry

    lax.fori_loop(0, IB, zrow, 0)
    for k in range(4):
        pltpu.sync_copy(zbuf.at[pl.ds(0, 128), :],
                        acc.at[pl.ds(s * NPT + k * 128, 128), :])
    pltpu.sync_copy(zbuf.at[pl.ds(0, 112), :],
                    acc.at[pl.ds(s * NPT + 512, 112), :])

    @pl.when(s == 15)
    def _():
        pltpu.sync_copy(zbuf.at[pl.ds(0, 16), :],
                        acc.at[pl.ds(16 * NPT, 16), :])


def _accumulate(h_ref, ei_ref, sidx, didx, rows, gsems, ssems, acc,
                base, nrows):
    ngrp = CH // NBUF

    def chunk(ci, carry):
        cbase = base + ci * CH
        pltpu.sync_copy(ei_ref.at[0, pl.ds(cbase, CH), :], sidx)
        pltpu.sync_copy(ei_ref.at[1, pl.ds(cbase, CH), :], didx)

        for b in range(NBUF):
            pltpu.async_copy(h_ref.at[sidx.at[b]], rows[b], gsems[b])

        def group(g, c2):
            t0 = g * NBUF
            for b in range(NBUF):
                pltpu.make_async_copy(h_ref.at[sidx.at[t0 + b]], rows[b],
                                      gsems[b]).wait()
                pltpu.async_copy(rows[b], acc.at[didx.at[t0 + b]], ssems[b],
                                 add=True)
            for b in range(NBUF):
                pltpu.make_async_copy(rows[b], acc.at[didx.at[t0 + b]],
                                      ssems[b]).wait()

                @pl.when(g + 1 < ngrp)
                def _():
                    pltpu.async_copy(h_ref.at[sidx.at[t0 + NBUF + b]],
                                     rows[b], gsems[b])
            return c2

        lax.fori_loop(0, ngrp, group, 0)
        return carry

    lax.fori_loop(0, nrows // CH, chunk, 0)


def _make_spmm_featsplit(n_rel):
    fh = H // 2
    rpt = ROWS // 16

    def body(*refs):
        h_refs = refs[:2 * n_rel]
        ei_refs = refs[2 * n_rel:3 * n_rel]
        out_ref = refs[3 * n_rel]
        sidx = refs[3 * n_rel + 1]
        didx = refs[3 * n_rel + 2]
        rows = refs[3 * n_rel + 3:3 * n_rel + 3 + NBUF]
        acc = refs[3 * n_rel + 3 + NBUF]
        gsems = refs[3 * n_rel + 4 + NBUF:3 * n_rel + 4 + 2 * NBUF]
        ssems = refs[3 * n_rel + 4 + 2 * NBUF:]

        c = lax.axis_index("c")
        s = lax.axis_index("s")

        _clear_acc(rows[0], acc, s)
        plsc.subcore_barrier()

        @pl.when(c == 0)
        def _():
            for r in range(n_rel):
                _accumulate(h_refs[2 * r], ei_refs[r], sidx, didx, rows,
                            gsems, ssems, acc, s * rpt, rpt)

        @pl.when(c == 1)
        def _():
            for r in range(n_rel):
                _accumulate(h_refs[2 * r + 1], ei_refs[r], sidx, didx, rows,
                            gsems, ssems, acc, s * rpt, rpt)

        plsc.subcore_barrier()
        pltpu.sync_copy(acc.at[pl.ds(s * NPT, NPT), :],
                        out_ref.at[pl.ds(s * NPT, NPT), pl.ds(c * fh, fh)])

        @pl.when(s == 15)
        def _():
            pltpu.sync_copy(acc.at[pl.ds(16 * NPT, 16), :],
                            out_ref.at[pl.ds(16 * NPT, 16), pl.ds(c * fh, fh)])

    return pl.kernel(
        body,
        out_type=jax.ShapeDtypeStruct((N, H), jnp.float32),
        mesh=_MESH,
        scratch_types=(
            [pltpu.VMEM((CH, IB), jnp.int32)] * 2 +
            [pltpu.VMEM((IB, fh), jnp.float32)] * NBUF +
            [pltpu.VMEM_SHARED((NA, fh), jnp.float32)] +
            [pltpu.SemaphoreType.DMA] * (2 * NBUF)
        ),
    )


def _make_spmm_edgesplit(n_rel):
    rpw = ROWS // 32

    def body(*refs):
        h_refs = refs[:n_rel]
        ei_refs = refs[n_rel:2 * n_rel]
        out_refs = refs[2 * n_rel:2 * n_rel + 2]
        sidx = refs[2 * n_rel + 2]
        didx = refs[2 * n_rel + 3]
        rows = refs[2 * n_rel + 4:2 * n_rel + 4 + NBUF]
        acc = refs[2 * n_rel + 4 + NBUF]
        gsems = refs[2 * n_rel + 5 + NBUF:2 * n_rel + 5 + 2 * NBUF]
        ssems = refs[2 * n_rel + 5 + 2 * NBUF:]

        c = lax.axis_index("c")
        s = lax.axis_index("s")
        w = c * 16 + s

        _clear_acc(rows[0], acc, s)
        plsc.subcore_barrier()

        for r in range(n_rel):
            _accumulate(h_refs[r], ei_refs[r], sidx, didx, rows,
                        gsems, ssems, acc, w * rpw, rpw)

        plsc.subcore_barrier()

        @pl.when(c == 0)
        def _():
            pltpu.sync_copy(acc.at[pl.ds(s * NPT, NPT), :],
                            out_refs[0].at[pl.ds(s * NPT, NPT), :])

            @pl.when(s == 15)
            def _():
                pltpu.sync_copy(acc.at[pl.ds(16 * NPT, 16), :],
                                out_refs[0].at[pl.ds(16 * NPT, 16), :])

        @pl.when(c == 1)
        def _():
            pltpu.sync_copy(acc.at[pl.ds(s * NPT, NPT), :],
                            out_refs[1].at[pl.ds(s * NPT, NPT), :])

            @pl.when(s == 15)
            def _():
                pltpu.sync_copy(acc.at[pl.ds(16 * NPT, 16), :],
                                out_refs[1].at[pl.ds(16 * NPT, 16), :])

    part = jax.ShapeDtypeStruct((N, OUT), jnp.float32)
    return pl.kernel(
        body,
        out_type=[part, part],
        mesh=_MESH,
        scratch_types=(
            [pltpu.VMEM((CH, IB), jnp.int32)] * 2 +
            [pltpu.VMEM((IB, OUT), jnp.float32)] * NBUF +
            [pltpu.VMEM_SHARED((NA, OUT), jnp.float32)] +
            [pltpu.SemaphoreType.DMA] * (2 * NBUF)
        ),
    )


def _pad_edges(ei):
    r = jnp.arange(EPAD, dtype=jnp.int32)
    pad = jnp.stack([r % N, N + (r % (NA - N))])
    return jnp.concatenate([ei.astype(jnp.int32), pad], axis=1).reshape(
        2, ROWS, IB)


def kernel(x_user, x_item, W1_follows, W1_rates, W1_ratedby,
           W2_follows, W2_rates, W2_ratedby,
           ei_follows, ei_rates, ei_ratedby):
    ei_f = _pad_edges(ei_follows)
    ei_r = _pad_edges(ei_rates)
    ei_rb = _pad_edges(ei_ratedby)

    mm1a = _make_mm1a()
    mm1b = _make_mm1b()
    mm2a = _make_mm2a()
    mm2b = _make_mm2b()
    spmm2_h = _make_spmm_featsplit(2)
    spmm1_h = _make_spmm_featsplit(1)
    spmm2_o = _make_spmm_edgesplit(2)
    spmm1_o = _make_spmm_edgesplit(1)
    merge_relu = _make_merge_relu()

    hf_lo, hf_hi, hrb_lo, hrb_hi = mm1a(x_user, x_item, W1_follows,
                                        W1_ratedby)
    hr_lo, hr_hi = mm1b(x_user, W1_rates)
    hu1 = spmm2_h(hf_lo, hf_hi, hrb_lo, hrb_hi, ei_f, ei_rb)
    dep1 = (hu1[0, 0] * 0.0).astype(jnp.int32)
    hi1 = spmm1_h(hr_lo, hr_hi, ei_r + dep1)

    gf, gr = mm2a(hu1, W2_follows, W2_rates)
    (grb,) = mm2b(hi1, W2_ratedby)
    hu2_p0, hu2_p1 = spmm2_o(gf, grb, ei_f, ei_rb)
    dep2 = (hu2_p0[0, 0] * 0.0).astype(jnp.int32)
    hi2_p0, hi2_p1 = spmm1_o(gr, ei_r + dep2)

    return merge_relu(hu2_p0, hu2_p1, hi2_p0, hi2_p1)


# speedup vs baseline: 5.2214x; 1.0477x over previous
import jax
import jax.numpy as jnp
from jax import lax
from jax.experimental import pallas as pl
from jax.experimental.pallas import tpu as pltpu
from jax.experimental.pallas import tpu_sc as plsc

N = 10000
E = 160000
H = 256
OUT = 128
IB = 128
ROWS = 1280
EPAD = ROWS * IB - E
NA = N + 16
NPT = 624
NBUF = 2
NBUF2 = 4
CH = 40


def _mm1a_body(xu_ref, xi_ref, wf_ref, wrb_ref,
               f_lo, f_hi, rb_lo, rb_hi):
    fh = H // 2
    mf = jnp.dot(xu_ref[...], wf_ref[...], preferred_element_type=jnp.float32)
    mrb = jnp.dot(xi_ref[...], wrb_ref[...],
                  preferred_element_type=jnp.float32)
    f_lo[...] = mf[:, :fh]
    f_hi[...] = mf[:, fh:]
    rb_lo[...] = mrb[:, :fh]
    rb_hi[...] = mrb[:, fh:]


def _make_mm1a():
    bm = 1000
    half = jax.ShapeDtypeStruct((N, H // 2), jnp.float32)
    return pl.pallas_call(
        _mm1a_body,
        grid=(N // bm,),
        in_specs=[
            pl.BlockSpec((bm, H), lambda i: (i, 0)),
            pl.BlockSpec((bm, H), lambda i: (i, 0)),
            pl.BlockSpec((H, H), lambda i: (0, 0)),
            pl.BlockSpec((H, H), lambda i: (0, 0)),
        ],
        out_specs=[pl.BlockSpec((bm, H // 2), lambda i: (i, 0))] * 4,
        out_shape=[half] * 4,
        compiler_params=pltpu.CompilerParams(
            dimension_semantics=("parallel",)),
    )


def _mm1b_body(xu_ref, wr_ref, r_lo, r_hi):
    fh = H // 2
    mr = jnp.dot(xu_ref[...], wr_ref[...], preferred_element_type=jnp.float32)
    r_lo[...] = mr[:, :fh]
    r_hi[...] = mr[:, fh:]


def _make_mm1b():
    bm = 1000
    half = jax.ShapeDtypeStruct((N, H // 2), jnp.float32)
    return pl.pallas_call(
        _mm1b_body,
        grid=(N // bm,),
        in_specs=[
            pl.BlockSpec((bm, H), lambda i: (i, 0)),
            pl.BlockSpec((H, H), lambda i: (0, 0)),
        ],
        out_specs=[pl.BlockSpec((bm, H // 2), lambda i: (i, 0))] * 2,
        out_shape=[half] * 2,
        compiler_params=pltpu.CompilerParams(
            dimension_semantics=("parallel",)),
    )


def _mm2a_body(xu_ref, wf_ref, wr_ref, f_lo, f_hi, r_lo, r_hi):
    fh = OUT // 2
    xu = jnp.maximum(xu_ref[...], 0.0)
    mf = jnp.dot(xu, wf_ref[...], preferred_element_type=jnp.float32)
    mr = jnp.dot(xu, wr_ref[...], preferred_element_type=jnp.float32)
    f_lo[...] = mf[:, :fh]
    f_hi[...] = mf[:, fh:]
    r_lo[...] = mr[:, :fh]
    r_hi[...] = mr[:, fh:]


def _make_mm2a():
    bm = 1000
    quarter = jax.ShapeDtypeStruct((N, OUT // 2), jnp.float32)
    return pl.pallas_call(
        _mm2a_body,
        grid=(N // bm,),
        in_specs=[
            pl.BlockSpec((bm, H), lambda i: (i, 0)),
            pl.BlockSpec((H, OUT), lambda i: (0, 0)),
            pl.BlockSpec((H, OUT), lambda i: (0, 0)),
        ],
        out_specs=[pl.BlockSpec((bm, OUT // 2), lambda i: (i, 0))] * 4,
        out_shape=[quarter] * 4,
        compiler_params=pltpu.CompilerParams(
            dimension_semantics=("parallel",)),
    )


def _mm2b_body(xi_ref, wrb_ref, rb_lo, rb_hi):
    fh = OUT // 2
    xi = jnp.maximum(xi_ref[...], 0.0)
    mrb = jnp.dot(xi, wrb_ref[...], preferred_element_type=jnp.float32)
    rb_lo[...] = mrb[:, :fh]
    rb_hi[...] = mrb[:, fh:]


def _make_mm2b():
    bm = 1000
    quarter = jax.ShapeDtypeStruct((N, OUT // 2), jnp.float32)
    return pl.pallas_call(
        _mm2b_body,
        grid=(N // bm,),
        in_specs=[
            pl.BlockSpec((bm, H), lambda i: (i, 0)),
            pl.BlockSpec((H, OUT), lambda i: (0, 0)),
        ],
        out_specs=[pl.BlockSpec((bm, OUT // 2), lambda i: (i, 0))] * 2,
        out_shape=[quarter] * 2,
        compiler_params=pltpu.CompilerParams(
            dimension_semantics=("parallel",)),
    )


def _merge_relu_body(ul_ref, uh_ref, il_ref, ih_ref, ou_ref, oi_ref):
    ou_ref[...] = jnp.maximum(
        jnp.concatenate([ul_ref[...], uh_ref[...]], axis=1), 0.0)
    oi_ref[...] = jnp.maximum(
        jnp.concatenate([il_ref[...], ih_ref[...]], axis=1), 0.0)


def _make_merge_relu():
    bm = 1000
    shp = jax.ShapeDtypeStruct((N, OUT), jnp.float32)
    qspec = pl.BlockSpec((bm, OUT // 2), lambda i: (i, 0))
    return pl.pallas_call(
        _merge_relu_body,
        grid=(N // bm,),
        in_specs=[qspec] * 4,
        out_specs=[pl.BlockSpec((bm, OUT), lambda i: (i, 0))] * 2,
        out_shape=[shp, shp],
        compiler_params=pltpu.CompilerParams(
            dimension_semantics=("parallel",)),
    )


_MESH = plsc.VectorSubcoreMesh(core_axis_name="c", subcore_axis_name="s",
                               num_cores=2)


def _clear_accs(zbuf, accs, s, w):
    zero = jnp.zeros((16,), jnp.float32)

    def zrow(r, carry):
        for j in range(w // 16):
            zbuf[r, pl.ds(16 * j, 16)] = zero
        return carry

    lax.fori_loop(0, IB, zrow, 0)
    for acc in accs:
        for k in range(4):
            pltpu.sync_copy(zbuf.at[pl.ds(0, 128), :],
                            acc.at[pl.ds(s * NPT + k * 128, 128), :])
        pltpu.sync_copy(zbuf.at[pl.ds(0, 112), :],
                        acc.at[pl.ds(s * NPT + 512, 112), :])

        @pl.when(s == 15)
        def _():
            pltpu.sync_copy(zbuf.at[pl.ds(0, 16), :],
                            acc.at[pl.ds(16 * NPT, 16), :])


def _accumulate(h_ref, ei_ref, sidx, didx, rows, gsems, ssems, acc,
                base, nrows):
    nb = len(rows)
    ngrp = CH // nb

    def chunk(ci, carry):
        cbase = base + ci * CH
        pltpu.sync_copy(ei_ref.at[0, pl.ds(cbase, CH), :], sidx)
        pltpu.sync_copy(ei_ref.at[1, pl.ds(cbase, CH), :], didx)

        for b in range(nb):
            pltpu.async_copy(h_ref.at[sidx.at[b]], rows[b], gsems[b])

        def group(g, c2):
            t0 = g * nb
            for b in range(nb):
                pltpu.make_async_copy(h_ref.at[sidx.at[t0 + b]], rows[b],
                                      gsems[b]).wait()
                pltpu.async_copy(rows[b], acc.at[didx.at[t0 + b]], ssems[b],
                                 add=True)
            for b in range(nb):
                pltpu.make_async_copy(rows[b], acc.at[didx.at[t0 + b]],
                                      ssems[b]).wait()

                @pl.when(g + 1 < ngrp)
                def _():
                    pltpu.async_copy(h_ref.at[sidx.at[t0 + nb + b]],
                                     rows[b], gsems[b])
            return c2

        lax.fori_loop(0, ngrp, group, 0)
        return carry

    lax.fori_loop(0, nrows // CH, chunk, 0)


def _make_spmm_featsplit(n_rel):
    fh = H // 2
    rpt = ROWS // 16

    def body(*refs):
        h_refs = refs[:2 * n_rel]
        ei_refs = refs[2 * n_rel:3 * n_rel]
        out_ref = refs[3 * n_rel]
        sidx = refs[3 * n_rel + 1]
        didx = refs[3 * n_rel + 2]
        rows = refs[3 * n_rel + 3:3 * n_rel + 3 + NBUF]
        acc = refs[3 * n_rel + 3 + NBUF]
        gsems = refs[3 * n_rel + 4 + NBUF:3 * n_rel + 4 + 2 * NBUF]
        ssems = refs[3 * n_rel + 4 + 2 * NBUF:]

        c = lax.axis_index("c")
        s = lax.axis_index("s")

        _clear_accs(rows[0], [acc], s, 128)
        plsc.subcore_barrier()

        @pl.when(c == 0)
        def _():
            for r in range(n_rel):
                _accumulate(h_refs[2 * r], ei_refs[r], sidx, didx, rows,
                            gsems, ssems, acc, s * rpt, rpt)

        @pl.when(c == 1)
        def _():
            for r in range(n_rel):
                _accumulate(h_refs[2 * r + 1], ei_refs[r], sidx, didx, rows,
                            gsems, ssems, acc, s * rpt, rpt)

        plsc.subcore_barrier()
        pltpu.sync_copy(acc.at[pl.ds(s * NPT, NPT), :],
                        out_ref.at[pl.ds(s * NPT, NPT), pl.ds(c * fh, fh)])

        @pl.when(s == 15)
        def _():
            pltpu.sync_copy(acc.at[pl.ds(16 * NPT, 16), :],
                            out_ref.at[pl.ds(16 * NPT, 16), pl.ds(c * fh, fh)])

    return pl.kernel(
        body,
        out_type=jax.ShapeDtypeStruct((N, H), jnp.float32),
        mesh=_MESH,
        scratch_types=(
            [pltpu.VMEM((CH, IB), jnp.int32)] * 2 +
            [pltpu.VMEM((IB, fh), jnp.float32)] * NBUF +
            [pltpu.VMEM_SHARED((NA, fh), jnp.float32)] +
            [pltpu.SemaphoreType.DMA] * (2 * NBUF)
        ),
    )


def _make_spmm_l2():
    fh = OUT // 2
    rpt = ROWS // 16

    def body(gf_lo, gf_hi, gr_lo, gr_hi, grb_lo, grb_hi,
             ei_f, ei_r, ei_rb, ou_lo, ou_hi, oi_lo, oi_hi,
             sidx, didx, *rest):
        rows = rest[:NBUF2]
        acc_u = rest[NBUF2]
        acc_i = rest[NBUF2 + 1]
        gsems = rest[NBUF2 + 2:NBUF2 + 2 + NBUF2]
        ssems = rest[NBUF2 + 2 + NBUF2:]

        c = lax.axis_index("c")
        s = lax.axis_index("s")

        _clear_accs(rows[0], [acc_u, acc_i], s, fh)
        plsc.subcore_barrier()

        def run(gf, gr, grb):
            _accumulate(gf, ei_f, sidx, didx, rows, gsems, ssems, acc_u,
                        s * rpt, rpt)
            _accumulate(grb, ei_rb, sidx, didx, rows, gsems, ssems, acc_u,
                        s * rpt, rpt)
            _accumulate(gr, ei_r, sidx, didx, rows, gsems, ssems, acc_i,
                        s * rpt, rpt)

        @pl.when(c == 0)
        def _():
            run(gf_lo, gr_lo, grb_lo)

        @pl.when(c == 1)
        def _():
            run(gf_hi, gr_hi, grb_hi)

        plsc.subcore_barrier()

        def wout(acc, o_lo, o_hi):
            @pl.when(c == 0)
            def _():
                pltpu.sync_copy(acc.at[pl.ds(s * NPT, NPT), :],
                                o_lo.at[pl.ds(s * NPT, NPT), :])

                @pl.when(s == 15)
                def _():
                    pltpu.sync_copy(acc.at[pl.ds(16 * NPT, 16), :],
                                    o_lo.at[pl.ds(16 * NPT, 16), :])

            @pl.when(c == 1)
            def _():
                pltpu.sync_copy(acc.at[pl.ds(s * NPT, NPT), :],
                                o_hi.at[pl.ds(s * NPT, NPT), :])

                @pl.when(s == 15)
                def _():
                    pltpu.sync_copy(acc.at[pl.ds(16 * NPT, 16), :],
                                    o_hi.at[pl.ds(16 * NPT, 16), :])

        wout(acc_u, ou_lo, ou_hi)
        wout(acc_i, oi_lo, oi_hi)

    quarter = jax.ShapeDtypeStruct((N, fh), jnp.float32)
    return pl.kernel(
        body,
        out_type=[quarter] * 4,
        mesh=_MESH,
        compiler_params=pltpu.CompilerParams(use_tc_tiling_on_sc=False),
        scratch_types=(
            [pltpu.VMEM((CH, IB), jnp.int32)] * 2 +
            [pltpu.VMEM((IB, fh), jnp.float32)] * NBUF2 +
            [pltpu.VMEM_SHARED((NA, fh), jnp.float32)] * 2 +
            [pltpu.SemaphoreType.DMA] * (2 * NBUF2)
        ),
    )


def _pad_edges(ei):
    r = jnp.arange(EPAD, dtype=jnp.int32)
    pad = jnp.stack([r % N, N + (r % (NA - N))])
    return jnp.concatenate([ei.astype(jnp.int32), pad], axis=1).reshape(
        2, ROWS, IB)


def kernel(x_user, x_item, W1_follows, W1_rates, W1_ratedby,
           W2_follows, W2_rates, W2_ratedby,
           ei_follows, ei_rates, ei_ratedby):
    ei_f = _pad_edges(ei_follows)
    ei_r = _pad_edges(ei_rates)
    ei_rb = _pad_edges(ei_ratedby)

    mm1a = _make_mm1a()
    mm1b = _make_mm1b()
    mm2a = _make_mm2a()
    mm2b = _make_mm2b()
    spmm2_h = _make_spmm_featsplit(2)
    spmm1_h = _make_spmm_featsplit(1)
    spmm_l2 = _make_spmm_l2()
    merge_relu = _make_merge_relu()

    hf_lo, hf_hi, hrb_lo, hrb_hi = mm1a(x_user, x_item, W1_follows,
                                        W1_ratedby)
    hr_lo, hr_hi = mm1b(x_user, W1_rates)
    hu1 = spmm2_h(hf_lo, hf_hi, hrb_lo, hrb_hi, ei_f, ei_rb)
    dep1 = (hu1[0, 0] * 0.0).astype(jnp.int32)
    hi1 = spmm1_h(hr_lo, hr_hi, ei_r + dep1)

    gf_lo, gf_hi, gr_lo, gr_hi = mm2a(hu1, W2_follows, W2_rates)
    grb_lo, grb_hi = mm2b(hi1, W2_ratedby)
    hu2_lo, hu2_hi, hi2_lo, hi2_hi = spmm_l2(
        gf_lo, gf_hi, gr_lo, gr_hi, grb_lo, grb_hi, ei_f, ei_r, ei_rb)

    return merge_relu(hu2_lo, hu2_hi, hi2_lo, hi2_hi)
